# baseline (speedup 1.0000x reference)
import functools

import jax
import jax.numpy as jnp
from jax import lax
from jax.experimental import pallas as pl
from jax.experimental.pallas import tpu as pltpu
from jax.experimental.pallas import tpu_sc as plsc

N = 50000
D = 64
E = 800000

NW = 32
RPW = 1568
NPAD = NW * RPW
MULT = 85599
SHIFT = 27
PCH = 2048
CAP = 13 * PCH
NBUCK = NW * NW
TRASH = NBUCK * CAP
PAIRS_SZ = TRASH + NW * 64
E_PAD = E + PCH
SENT = RPW << 16

_mesh = plsc.VectorSubcoreMesh(core_axis_name="c", subcore_axis_name="s")


def _wid():
  return lax.axis_index("s") * 2 + lax.axis_index("c")


def _take(x, idx):
  return jnp.take_along_axis(
      x, idx, axis=0, mode=lax.GatherScatterMode.PROMISE_IN_BOUNDS
  )


def _bin_body(src_hbm, dst_hbm, pairs_hbm, counts_hbm,
              srcv, dstv, stagepos, stageval, padpos, padval, cnt, sem):
  w = _wid()
  iota = lax.iota(jnp.int32, 16)
  cnt[pl.ds(0, 16)] = jnp.zeros((16,), jnp.int32)
  cnt[pl.ds(16, 16)] = jnp.zeros((16,), jnp.int32)

  nv = jnp.where(w < 16, 1563, 1562)
  start_edge = (w * 1562 + jnp.minimum(w, 16)) * 16
  nedges = nv * 16
  nch = (nedges + PCH - 1) // PCH

  def chunk_body(j, _):
    base = start_edge + j * PCH
    pltpu.sync_copy(src_hbm.at[pl.ds(base, PCH)], srcv)
    pltpu.sync_copy(dst_hbm.at[pl.ds(base, PCH)], dstv)
    m = jnp.minimum(PCH, nedges - j * PCH)
    mv = m // 16

    def vreg_body(v, _):
      sl = pl.ds(v * 16, 16)
      dvec = dstv[sl]
      svec = srcv[sl]
      du = dvec.astype(jnp.uint32)
      b = (du * jnp.uint32(MULT) >> SHIFT).astype(jnp.int32)
      loc = dvec - b * RPW
      val = svec | (loc << 16)
      occ, lastm = plsc.scan_count(b)
      rank = occ - 1
      basec = plsc.load_gather(cnt, [b])
      pos = basec + rank
      plsc.store_scatter(cnt, [b], pos + 1, mask=lastm)
      r = v // 8
      csl = pl.ds((v % 8) * 16, 16)
      stagepos[r, csl] = (w * NW + b) * CAP + pos
      stageval[r, csl] = val
      return 0

    lax.fori_loop(0, mv, vreg_body, 0)

    def trash_body(v, _):
      r = v // 8
      csl = pl.ds((v % 8) * 16, 16)
      stagepos[r, csl] = TRASH + w * 64 + iota
      return 0

    lax.fori_loop(mv, PCH // 16, trash_body, 0)

    copies = [
        pltpu.async_copy(stageval.at[r], pairs_hbm.at[stagepos.at[r]], sem)
        for r in range(16)
    ]
    for cp in copies:
      cp.wait()
    return 0

  lax.fori_loop(0, nch, chunk_body, 0)

  for g in range(2):
    bv = iota + 16 * g
    cv = cnt[pl.ds(16 * g, 16)]
    for k in range(16):
      fl = g * 16 + k
      r = fl // 4
      csl = pl.ds((fl % 4) * 16, 16)
      padpos[r, csl] = (w * NW + bv) * CAP + cv + k
      padval[r, csl] = jnp.full((16,), SENT, jnp.int32)
  copies = [
      pltpu.async_copy(padval.at[r], pairs_hbm.at[padpos.at[r]], sem)
      for r in range(8)
  ]
  for cp in copies:
    cp.wait()

  pltpu.sync_copy(cnt, counts_hbm.at[pl.ds(w * NW, NW)])


@jax.jit
def _bin_edges(src, dst):
  f = pl.kernel(
      _bin_body,
      out_type=[
          jax.ShapeDtypeStruct((PAIRS_SZ,), jnp.int32),
          jax.ShapeDtypeStruct((NW * NW,), jnp.int32),
      ],
      mesh=_mesh,
      compiler_params=pltpu.CompilerParams(needs_layout_passes=False),
      scratch_types=[
          pltpu.VMEM((PCH,), jnp.int32),
          pltpu.VMEM((PCH,), jnp.int32),
          pltpu.VMEM((16, 128), jnp.int32),
          pltpu.VMEM((16, 128), jnp.int32),
          pltpu.VMEM((8, 64), jnp.int32),
          pltpu.VMEM((8, 64), jnp.int32),
          pltpu.VMEM((NW,), jnp.int32),
          pltpu.SemaphoreType.DMA,
      ],
  )
  return f(src, dst)


GB = 64


def _segmax_body(hp_hbm, pairs_hbm, counts_hbm, hn_hbm,
                 acc0, acc1, acc2, acc3, pairs_v, srcidx, locb,
                 rows0, rows1, counts_v, sem0, sem1):
  w = _wid()
  iota = lax.iota(jnp.int32, 16)
  accs = (acc0, acc1, acc2, acc3)
  pltpu.sync_copy(counts_hbm, counts_v.at[pl.ds(0, NW * NW)])

  def zero_body(r, _):
    for g in range(4):
      accs[g][pl.ds(r * 16, 16)] = jnp.zeros((16,), jnp.float32)
    return 0

  lax.fori_loop(0, RPW + 1, zero_body, 0, unroll=4)

  rows_bufs = (rows0, rows1)
  sems = (sem0, sem1)

  def t_body(t, _):
    cnt = counts_v[pl.ds(t * NW + w, 16)][0]
    cntp = ((cnt + 15) // 16) * 16
    bucket = (t * NW + w) * CAP
    nch = (cntp + PCH - 1) // PCH

    def idx_slice(sb):
      return srcidx.at[sb // 2, pl.ds((sb % 2) * GB, GB)]

    def issue(sb, buf):
      pltpu.async_copy(hp_hbm.at[idx_slice(sb)], rows_bufs[buf], sems[buf])

    def drain(sb, buf):
      pltpu.make_async_copy(
          hp_hbm.at[idx_slice(sb)], rows_bufs[buf], sems[buf]
      ).wait()

    def chunk_body(j, _):
      pltpu.sync_copy(pairs_hbm.at[pl.ds(bucket + j * PCH, PCH)], pairs_v)
      m = jnp.minimum(PCH, cntp - j * PCH)

      def unpack_body(v, _):
        sl = pl.ds(v * 16, 16)
        pv = pairs_v[sl]
        valid = (j * PCH + v * 16 + iota) < cntp
        srcidx[v // 8, pl.ds((v % 8) * 16, 16)] = jnp.where(
            valid, pv & 0xFFFF, 0
        )
        locb[sl] = jnp.where(valid, pv >> 16, RPW) * 16
        return 0

      lax.fori_loop(0, PCH // 16, unpack_body, 0, unroll=4)
      nsub = (m + GB - 1) // GB

      @pl.when(nsub > 0)
      def _():
        issue(0, 0)

      def process(sb, buf):
        drain(sb, buf)

        @pl.when(sb + 1 < nsub)
        def _():
          issue(sb + 1, 1 - buf)

        rows = rows_bufs[buf]

        def ev_body(q, _):
          lvec = locb[pl.ds(sb * GB + q * 16, 16)]
          for k in range(16):
            r = q * 16 + k
            base = lvec[k]
            asl = pl.ds(base, 16)
            for g in range(4):
              accs[g][asl] = jnp.maximum(
                  accs[g][asl], rows[r, pl.ds(g * 16, 16)]
              )
          return 0

        lax.fori_loop(0, GB // 16, ev_body, 0)

      def pair_body(p, _):
        sb = p * 2

        @pl.when(sb < nsub)
        def _():
          process(sb, 0)

        @pl.when(sb + 1 < nsub)
        def _():
          process(sb + 1, 1)

        return 0

      lax.fori_loop(0, (nsub + 1) // 2, pair_body, 0)
      return 0

    lax.fori_loop(0, nch, chunk_body, 0)
    return 0

  lax.fori_loop(0, NW, t_body, 0)
  for g in range(4):
    pltpu.sync_copy(
        accs[g].at[pl.ds(0, RPW * 16)],
        hn_hbm.at[pl.ds(g * NPAD * 16 + w * RPW * 16, RPW * 16)],
    )


@jax.jit
def _segmax(hp, pairs, counts):
  f = pl.kernel(
      _segmax_body,
      out_type=jax.ShapeDtypeStruct((4 * NPAD * 16,), jnp.float32),
      mesh=_mesh,
      compiler_params=pltpu.CompilerParams(needs_layout_passes=False),
      scratch_types=[
          pltpu.VMEM(((RPW + 1) * 16,), jnp.float32),
          pltpu.VMEM(((RPW + 1) * 16,), jnp.float32),
          pltpu.VMEM(((RPW + 1) * 16,), jnp.float32),
          pltpu.VMEM(((RPW + 1) * 16,), jnp.float32),
          pltpu.VMEM((PCH,), jnp.int32),
          pltpu.VMEM((16, 128), jnp.int32),
          pltpu.VMEM((PCH,), jnp.int32),
          pltpu.VMEM((GB, 2 * D), jnp.float32),
          pltpu.VMEM((GB, 2 * D), jnp.float32),
          pltpu.VMEM((NW * NW + 16,), jnp.int32),
          pltpu.SemaphoreType.DMA,
          pltpu.SemaphoreType.DMA,
      ],
  )
  return f(hp, pairs, counts)


_BR = 512
_GRID = ((N + _BR - 1) // _BR,)


def _tc_first(x_ref, wp_ref, bp_ref, ws_ref, b_ref, hp_ref, s_ref):
  x = x_ref[...]
  hp = jax.nn.relu(
      jnp.dot(x, wp_ref[...], preferred_element_type=jnp.float32) + bp_ref[0, :]
  )
  hp_ref[...] = jnp.concatenate([hp, jnp.zeros((_BR, D), jnp.float32)], axis=1)
  s_ref[...] = (
      jnp.dot(x, ws_ref[...], preferred_element_type=jnp.float32) + b_ref[0, :]
  )


def _tc_mid(s_ref, hn_ref, wn_ref, wp_ref, bp_ref, ws_ref, b_ref,
            hp_ref, so_ref):
  h = jax.nn.relu(
      s_ref[...]
      + jnp.dot(hn_ref[...], wn_ref[...], preferred_element_type=jnp.float32)
  )
  hp = jax.nn.relu(
      jnp.dot(h, wp_ref[...], preferred_element_type=jnp.float32) + bp_ref[0, :]
  )
  hp_ref[...] = jnp.concatenate([hp, jnp.zeros((_BR, D), jnp.float32)], axis=1)
  so_ref[...] = (
      jnp.dot(h, ws_ref[...], preferred_element_type=jnp.float32) + b_ref[0, :]
  )


def _tc_last(s_ref, hn_ref, wn_ref, out_ref):
  out_ref[...] = s_ref[...] + jnp.dot(
      hn_ref[...], wn_ref[...], preferred_element_type=jnp.float32
  )


_row_spec = pl.BlockSpec((_BR, D), lambda i: (i, 0))
_hp_spec = pl.BlockSpec((_BR, 2 * D), lambda i: (i, 0))
_mat_spec = pl.BlockSpec((D, D), lambda i: (0, 0))
_vec_spec = pl.BlockSpec((1, D), lambda i: (0, 0))


@jax.jit
def _first_stage(x, wp, bp, ws, b):
  return pl.pallas_call(
      _tc_first,
      grid=_GRID,
      in_specs=[_row_spec, _mat_spec, _vec_spec, _mat_spec, _vec_spec],
      out_specs=[_hp_spec, _row_spec],
      out_shape=[
          jax.ShapeDtypeStruct((N, 2 * D), jnp.float32),
          jax.ShapeDtypeStruct((N, D), jnp.float32),
      ],
  )(x, wp, bp.reshape(1, D), ws, b.reshape(1, D))


@jax.jit
def _mid_stage(s, hn, wn, wp, bp, ws, b):
  return pl.pallas_call(
      _tc_mid,
      grid=_GRID,
      in_specs=[_row_spec, _row_spec, _mat_spec, _mat_spec, _vec_spec,
                _mat_spec, _vec_spec],
      out_specs=[_hp_spec, _row_spec],
      out_shape=[
          jax.ShapeDtypeStruct((N, 2 * D), jnp.float32),
          jax.ShapeDtypeStruct((N, D), jnp.float32),
      ],
  )(s, hn, wn, wp, bp.reshape(1, D), ws, b.reshape(1, D))


@jax.jit
def _last_stage(s, hn, wn):
  return pl.pallas_call(
      _tc_last,
      grid=_GRID,
      in_specs=[_row_spec, _row_spec, _mat_spec],
      out_specs=_row_spec,
      out_shape=jax.ShapeDtypeStruct((N, D), jnp.float32),
  )(s, hn, wn)


@jax.jit
def kernel(in_feat, edge_index, params):
  src = edge_index[0].astype(jnp.int32)
  dst = edge_index[1].astype(jnp.int32)
  src = jnp.pad(src, (0, E_PAD - E))
  dst = jnp.pad(dst, (0, E_PAD - E))
  p = params

  pairs, counts = _bin_edges(src, dst)

  hp, s = _first_stage(in_feat, p["W_pool0"], p["b_pool0"], p["W_self0"],
                       p["b0"])
  def unstrip(hn4):
    return (
        hn4.reshape(4, NPAD, 16).transpose(1, 0, 2).reshape(NPAD, D)[:N]
    )

  hn = unstrip(_segmax(hp, pairs, counts))
  hp, s = _mid_stage(s, hn, p["W_neigh0"], p["W_pool1"], p["b_pool1"],
                     p["W_self1"], p["b1"])
  hn = unstrip(_segmax(hp, pairs, counts))
  hp, s = _mid_stage(s, hn, p["W_neigh1"], p["W_pool2"], p["b_pool2"],
                     p["W_self2"], p["b2"])
  hn = unstrip(_segmax(hp, pairs, counts))
  return _last_stage(s, hn, p["W_neigh2"])


# speedup vs baseline: 1.1156x; 1.1156x over previous
import functools

import jax
import jax.numpy as jnp
from jax import lax
from jax.experimental import pallas as pl
from jax.experimental.pallas import tpu as pltpu
from jax.experimental.pallas import tpu_sc as plsc

N = 50000
D = 64
E = 800000

NW = 32
RPW = 1568
NPAD = NW * RPW
MULT = 85599
SHIFT = 27
PCH = 2048
CAP = 13 * PCH
NBUCK = NW * NW
TRASH = NBUCK * CAP
PAIRS_SZ = TRASH + NW * 64
E_PAD = E + PCH
SENT = RPW << 16

_mesh = plsc.VectorSubcoreMesh(core_axis_name="c", subcore_axis_name="s")


def _wid():
  return lax.axis_index("s") * 2 + lax.axis_index("c")


def _take(x, idx):
  return jnp.take_along_axis(
      x, idx, axis=0, mode=lax.GatherScatterMode.PROMISE_IN_BOUNDS
  )


NCH = 13


def _bin_body(src_hbm, dst_hbm, pairs_hbm, counts_hbm,
              srcv0, srcv1, dstv0, dstv1, sp0, sv0, sp1, sv1,
              padpos, padval, cnt, lsem0, lsem1, ssem0, ssem1):
  w = _wid()
  iota = lax.iota(jnp.int32, 16)
  cnt[pl.ds(0, 16)] = jnp.zeros((16,), jnp.int32)
  cnt[pl.ds(16, 16)] = jnp.zeros((16,), jnp.int32)

  nv = jnp.where(w < 16, 1563, 1562)
  start_edge = (w * 1562 + jnp.minimum(w, 16)) * 16
  nedges = nv * 16

  srcs, dsts = (srcv0, srcv1), (dstv0, dstv1)
  sps, svs = (sp0, sp1), (sv0, sv1)
  lsems, ssems = (lsem0, lsem1), (ssem0, ssem1)

  def issue_load(j, b):
    base = start_edge + j * PCH
    pltpu.async_copy(src_hbm.at[pl.ds(base, PCH)], srcs[b], lsems[b])
    pltpu.async_copy(dst_hbm.at[pl.ds(base, PCH)], dsts[b], lsems[b])

  def drain_load(j, b):
    base = start_edge + j * PCH
    pltpu.make_async_copy(
        src_hbm.at[pl.ds(base, PCH)], srcs[b], lsems[b]
    ).wait()
    pltpu.make_async_copy(
        dst_hbm.at[pl.ds(base, PCH)], dsts[b], lsems[b]
    ).wait()

  def fire_scatter(b):
    for r in range(16):
      pltpu.async_copy(svs[b].at[r], pairs_hbm.at[sps[b].at[r]], ssems[b])

  def drain_scatter(b):
    for r in range(16):
      pltpu.make_async_copy(
          svs[b].at[r], pairs_hbm.at[sps[b].at[r]], ssems[b]
      ).wait()

  def make_vreg_body(srcv, dstv, stagepos, stageval):
    def vreg_body(v, _):
      sl = pl.ds(v * 16, 16)
      dvec = dstv[sl]
      svec = srcv[sl]
      du = dvec.astype(jnp.uint32)
      b = (du * jnp.uint32(MULT) >> SHIFT).astype(jnp.int32)
      loc = dvec - b * RPW
      val = svec | (loc << 16)
      occ, lastm = plsc.scan_count(b)
      rank = occ - 1
      basec = plsc.load_gather(cnt, [b])
      pos = basec + rank
      plsc.store_scatter(cnt, [b], pos + 1, mask=lastm)
      r = v // 8
      csl = pl.ds((v % 8) * 16, 16)
      stagepos[r, csl] = (w * NW + b) * CAP + pos
      stageval[r, csl] = val
      return 0

    return vreg_body

  issue_load(0, 0)
  for j in range(NCH):
    b = j % 2
    drain_load(j, b)
    if j + 1 < NCH:
      issue_load(j + 1, 1 - b)
    if j >= 2:
      drain_scatter(b)
    if j + 1 < NCH:
      lax.fori_loop(0, PCH // 16, make_vreg_body(srcs[b], dsts[b],
                                                 sps[b], svs[b]), 0)
    else:
      mv = (nedges - j * PCH) // 16
      lax.fori_loop(0, mv, make_vreg_body(srcs[b], dsts[b],
                                          sps[b], svs[b]), 0)

      def trash_body(v, _):
        r = v // 8
        csl = pl.ds((v % 8) * 16, 16)
        sps[b][r, csl] = TRASH + w * 64 + iota
        return 0

      lax.fori_loop(mv, PCH // 16, trash_body, 0)
    fire_scatter(b)
  drain_scatter(1)
  drain_scatter(0)

  for g in range(2):
    bv = iota + 16 * g
    cv = cnt[pl.ds(16 * g, 16)]
    for k in range(16):
      fl = g * 16 + k
      r = fl // 4
      csl = pl.ds((fl % 4) * 16, 16)
      padpos[r, csl] = (w * NW + bv) * CAP + cv + k
      padval[r, csl] = jnp.full((16,), SENT, jnp.int32)
  copies = [
      pltpu.async_copy(padval.at[r], pairs_hbm.at[padpos.at[r]], ssem0)
      for r in range(8)
  ]
  for cp in copies:
    cp.wait()

  pltpu.sync_copy(cnt, counts_hbm.at[pl.ds(w * NW, NW)])


@jax.jit
def _bin_edges(src, dst):
  f = pl.kernel(
      _bin_body,
      out_type=[
          jax.ShapeDtypeStruct((PAIRS_SZ,), jnp.int32),
          jax.ShapeDtypeStruct((NW * NW,), jnp.int32),
      ],
      mesh=_mesh,
      compiler_params=pltpu.CompilerParams(needs_layout_passes=False),
      scratch_types=[
          pltpu.VMEM((PCH,), jnp.int32),
          pltpu.VMEM((PCH,), jnp.int32),
          pltpu.VMEM((PCH,), jnp.int32),
          pltpu.VMEM((PCH,), jnp.int32),
          pltpu.VMEM((16, 128), jnp.int32),
          pltpu.VMEM((16, 128), jnp.int32),
          pltpu.VMEM((16, 128), jnp.int32),
          pltpu.VMEM((16, 128), jnp.int32),
          pltpu.VMEM((8, 64), jnp.int32),
          pltpu.VMEM((8, 64), jnp.int32),
          pltpu.VMEM((NW,), jnp.int32),
          pltpu.SemaphoreType.DMA,
          pltpu.SemaphoreType.DMA,
          pltpu.SemaphoreType.DMA,
          pltpu.SemaphoreType.DMA,
      ],
  )
  return f(src, dst)


GB = 64


def _segmax_body(hp_hbm, pairs_hbm, counts_hbm, hn_hbm,
                 acc, pairs_v, srcidx, locb, rows0, rows1, counts_v,
                 sem0, sem1):
  w = _wid()
  iota = lax.iota(jnp.int32, 16)
  pltpu.sync_copy(counts_hbm, counts_v.at[pl.ds(0, NW * NW)])

  def zero_body(r, _):
    acc[pl.ds(r * 16, 16)] = jnp.zeros((16,), jnp.float32)
    return 0

  lax.fori_loop(0, (RPW + 1) * 4, zero_body, 0, unroll=8)

  rows_bufs = (rows0, rows1)
  sems = (sem0, sem1)

  def t_body(t, _):
    cnt = counts_v[pl.ds(t * NW + w, 16)][0]
    cntp = ((cnt + 15) // 16) * 16
    bucket = (t * NW + w) * CAP
    nch = (cntp + PCH - 1) // PCH

    def idx_slice(sb):
      return srcidx.at[sb // 2, pl.ds((sb % 2) * GB, GB)]

    def issue(sb, buf):
      pltpu.async_copy(hp_hbm.at[idx_slice(sb)], rows_bufs[buf], sems[buf])

    def drain(sb, buf):
      pltpu.make_async_copy(
          hp_hbm.at[idx_slice(sb)], rows_bufs[buf], sems[buf]
      ).wait()

    def chunk_body(j, _):
      pltpu.sync_copy(pairs_hbm.at[pl.ds(bucket + j * PCH, PCH)], pairs_v)
      m = jnp.minimum(PCH, cntp - j * PCH)

      def unpack_body(v, _):
        sl = pl.ds(v * 16, 16)
        pv = pairs_v[sl]
        valid = (j * PCH + v * 16 + iota) < cntp
        srcidx[v // 8, pl.ds((v % 8) * 16, 16)] = jnp.where(
            valid, pv & 0xFFFF, 0
        )
        locb[sl] = jnp.where(valid, pv >> 16, RPW) * D
        return 0

      lax.fori_loop(0, PCH // 16, unpack_body, 0, unroll=4)
      nsub = (m + GB - 1) // GB

      @pl.when(nsub > 0)
      def _():
        issue(0, 0)

      def process(sb, buf):
        drain(sb, buf)

        @pl.when(sb + 1 < nsub)
        def _():
          issue(sb + 1, 1 - buf)

        rows = rows_bufs[buf]

        def ev_body(q, _):
          lvec = locb[pl.ds(sb * GB + q * 16, 16)]
          for k in range(16):
            r = q * 16 + k
            base = lvec[k]
            for g in range(4):
              asl = pl.ds(base + g * 16, 16)
              acc[asl] = jnp.maximum(acc[asl], rows[r, pl.ds(g * 16, 16)])
          return 0

        lax.fori_loop(0, GB // 16, ev_body, 0)

      def pair_body(p, _):
        sb = p * 2

        @pl.when(sb < nsub)
        def _():
          process(sb, 0)

        @pl.when(sb + 1 < nsub)
        def _():
          process(sb + 1, 1)

        return 0

      lax.fori_loop(0, (nsub + 1) // 2, pair_body, 0)
      return 0

    lax.fori_loop(0, nch, chunk_body, 0)
    return 0

  lax.fori_loop(0, NW, t_body, 0)
  pltpu.sync_copy(
      acc.at[pl.ds(0, RPW * D)], hn_hbm.at[pl.ds(w * RPW * D, RPW * D)]
  )


@jax.jit
def _segmax(hp, pairs, counts):
  f = pl.kernel(
      _segmax_body,
      out_type=jax.ShapeDtypeStruct((NPAD * D,), jnp.float32),
      mesh=_mesh,
      compiler_params=pltpu.CompilerParams(needs_layout_passes=False),
      scratch_types=[
          pltpu.VMEM(((RPW + 1) * D,), jnp.float32),
          pltpu.VMEM((PCH,), jnp.int32),
          pltpu.VMEM((16, 128), jnp.int32),
          pltpu.VMEM((PCH,), jnp.int32),
          pltpu.VMEM((GB, 2 * D), jnp.float32),
          pltpu.VMEM((GB, 2 * D), jnp.float32),
          pltpu.VMEM((NW * NW + 16,), jnp.int32),
          pltpu.SemaphoreType.DMA,
          pltpu.SemaphoreType.DMA,
      ],
  )
  return f(hp, pairs, counts)


_BR = 512
_GRID = ((N + _BR - 1) // _BR,)


def _tc_first(x_ref, wp_ref, bp_ref, ws_ref, b_ref, hp_ref, s_ref):
  x = x_ref[...]
  hp = jax.nn.relu(
      jnp.dot(x, wp_ref[...], preferred_element_type=jnp.float32) + bp_ref[0, :]
  )
  hp_ref[...] = jnp.concatenate([hp, jnp.zeros((_BR, D), jnp.float32)], axis=1)
  s_ref[...] = (
      jnp.dot(x, ws_ref[...], preferred_element_type=jnp.float32) + b_ref[0, :]
  )


def _tc_mid(s_ref, hn_ref, wn_ref, wp_ref, bp_ref, ws_ref, b_ref,
            hp_ref, so_ref):
  h = jax.nn.relu(
      s_ref[...]
      + jnp.dot(hn_ref[...], wn_ref[...], preferred_element_type=jnp.float32)
  )
  hp = jax.nn.relu(
      jnp.dot(h, wp_ref[...], preferred_element_type=jnp.float32) + bp_ref[0, :]
  )
  hp_ref[...] = jnp.concatenate([hp, jnp.zeros((_BR, D), jnp.float32)], axis=1)
  so_ref[...] = (
      jnp.dot(h, ws_ref[...], preferred_element_type=jnp.float32) + b_ref[0, :]
  )


def _tc_last(s_ref, hn_ref, wn_ref, out_ref):
  out_ref[...] = s_ref[...] + jnp.dot(
      hn_ref[...], wn_ref[...], preferred_element_type=jnp.float32
  )


_row_spec = pl.BlockSpec((_BR, D), lambda i: (i, 0))
_hp_spec = pl.BlockSpec((_BR, 2 * D), lambda i: (i, 0))
_mat_spec = pl.BlockSpec((D, D), lambda i: (0, 0))
_vec_spec = pl.BlockSpec((1, D), lambda i: (0, 0))


@jax.jit
def _first_stage(x, wp, bp, ws, b):
  return pl.pallas_call(
      _tc_first,
      grid=_GRID,
      in_specs=[_row_spec, _mat_spec, _vec_spec, _mat_spec, _vec_spec],
      out_specs=[_hp_spec, _row_spec],
      out_shape=[
          jax.ShapeDtypeStruct((N, 2 * D), jnp.float32),
          jax.ShapeDtypeStruct((N, D), jnp.float32),
      ],
  )(x, wp, bp.reshape(1, D), ws, b.reshape(1, D))


@jax.jit
def _mid_stage(s, hn, wn, wp, bp, ws, b):
  return pl.pallas_call(
      _tc_mid,
      grid=_GRID,
      in_specs=[_row_spec, _row_spec, _mat_spec, _mat_spec, _vec_spec,
                _mat_spec, _vec_spec],
      out_specs=[_hp_spec, _row_spec],
      out_shape=[
          jax.ShapeDtypeStruct((N, 2 * D), jnp.float32),
          jax.ShapeDtypeStruct((N, D), jnp.float32),
      ],
  )(s, hn, wn, wp, bp.reshape(1, D), ws, b.reshape(1, D))


@jax.jit
def _last_stage(s, hn, wn):
  return pl.pallas_call(
      _tc_last,
      grid=_GRID,
      in_specs=[_row_spec, _row_spec, _mat_spec],
      out_specs=_row_spec,
      out_shape=jax.ShapeDtypeStruct((N, D), jnp.float32),
  )(s, hn, wn)


@jax.jit
def kernel(in_feat, edge_index, params):
  src = edge_index[0].astype(jnp.int32)
  dst = edge_index[1].astype(jnp.int32)
  src = jnp.pad(src, (0, E_PAD - E))
  dst = jnp.pad(dst, (0, E_PAD - E))
  p = params

  pairs, counts = _bin_edges(src, dst)

  hp, s = _first_stage(in_feat, p["W_pool0"], p["b_pool0"], p["W_self0"],
                       p["b0"])
  hn = _segmax(hp, pairs, counts).reshape(NPAD, D)[:N]
  hp, s = _mid_stage(s, hn, p["W_neigh0"], p["W_pool1"], p["b_pool1"],
                     p["W_self1"], p["b1"])
  hn = _segmax(hp, pairs, counts).reshape(NPAD, D)[:N]
  hp, s = _mid_stage(s, hn, p["W_neigh1"], p["W_pool2"], p["b_pool2"],
                     p["W_self2"], p["b2"])
  hn = _segmax(hp, pairs, counts).reshape(NPAD, D)[:N]
  return _last_stage(s, hn, p["W_neigh2"])
